# baseline (speedup 1.0000x reference)
import functools

import jax
import jax.numpy as jnp
from jax import lax
from jax.experimental import pallas as pl
from jax.experimental.pallas import tpu as pltpu
from jax.experimental.pallas import tpu_sc as plsc

VOCAB = 100000
HIDDEN = 768
BATCH = 4
SEQ = 2048
EPS = 1e-12

L = 16
NC, NS = 2, 16
NW = NC * NS
PPW = SEQ // NW
TPW = BATCH * PPW
CH = 16
NCHUNK = TPW // CH
QPC = PPW // CH
NVH = HIDDEN // L
SUB = 8

_DNUMS = lax.GatherDimensionNumbers(offset_dims=(), collapsed_slice_dims=(0,),
                                    start_index_map=(0,))


def _xlane_sum(v):
    idx = lax.iota(jnp.int32, L)
    for sh in (1, 2, 4, 8):
        perm = jnp.bitwise_xor(idx, sh)
        v = v + lax.gather(v, perm[:, None], _DNUMS, slice_sizes=(1,),
                           mode=lax.GatherScatterMode.PROMISE_IN_BOUNDS)
    return v


def _rsqrt(x):
    i = lax.bitcast_convert_type(x, jnp.int32)
    i = jnp.int32(0x5F3759DF) - lax.shift_right_arithmetic(i, 1)
    y = lax.bitcast_convert_type(i, jnp.float32)
    for _ in range(3):
        y = y * (1.5 - 0.5 * x * y * y)
    return y


def _sc_body(ids_hbm, tt_hbm, word_hbm, pos_hbm, typ_hbm, scale_hbm, bias_hbm,
             out_hbm, idx_all, tt_all, buf0, buf1, ptab, tbuf, sv, bv,
             ssem, gs0, gs1, ws0, ws1):
    wid = lax.axis_index("s") * NC + lax.axis_index("c")
    pbase = wid * PPW
    bufs = (buf0, buf1)
    gsems = (gs0, gs1)

    stage = []
    for b in range(BATCH):
        for q in range(QPC):
            c = QPC * b + q
            stage.append(pltpu.async_copy(
                ids_hbm.at[pl.ds(b * SEQ + pbase + q * CH, CH)],
                idx_all.at[c], ssem))
        stage.append(pltpu.async_copy(
            tt_hbm.at[pl.ds(b * SEQ + pbase, PPW)],
            tt_all.at[pl.ds(b * PPW, PPW)], ssem))
    stage.append(pltpu.async_copy(scale_hbm, sv, ssem))
    stage.append(pltpu.async_copy(bias_hbm, bv, ssem))
    stage.append(pltpu.async_copy(typ_hbm, tbuf, ssem))
    stage.append(pltpu.async_copy(pos_hbm.at[pl.ds(pbase, PPW)],
                                  ptab.at[0], ssem))
    for h in stage:
        h.wait()

    pltpu.async_copy(word_hbm.at[idx_all.at[0]], buf0, gs0)

    def prep_j(j, _):
        sl = pl.ds(j * L, L)
        t0 = tbuf[0, sl]
        d = tbuf[1, sl] - t0

        def prep_i(i, _):
            v0 = ptab[0, i, sl] + t0
            ptab[0, i, sl] = v0
            ptab[1, i, sl] = v0 + d
            return 0
        lax.fori_loop(0, PPW, prep_i, 0)
        return 0
    lax.fori_loop(0, NVH, prep_j, 0)

    zero = jnp.zeros((L,), jnp.float32)

    def compute_chunk(buf, c, poff):
        tt16 = tt_all[pl.ds(c * CH, CH)]
        for l in range(0, CH, 4):
            ii = [l + t for t in range(4)]
            prow = [poff + i for i in ii]
            ttk = [tt16[i] for i in ii]

            @plsc.parallel_loop(0, NVH, step=1, unroll=SUB,
                                carry=(zero,) * 8)
            def stats(j, carry):
                sl = pl.ds(j * L, L)
                out = []
                for t in range(4):
                    v = buf[ii[t], sl] + ptab[ttk[t], prow[t], sl]
                    buf[ii[t], sl] = v
                    out.append(carry[2 * t] + v)
                    out.append(carry[2 * t + 1] + v * v)
                return tuple(out)
            accs = stats

            means = [_xlane_sum(accs[2 * t]) * (1.0 / HIDDEN)
                     for t in range(4)]
            msqs = [_xlane_sum(accs[2 * t + 1]) * (1.0 / HIDDEN)
                    for t in range(4)]
            rstds = [_rsqrt(msqs[t] - means[t] * means[t] + EPS)
                     for t in range(4)]

            @plsc.parallel_loop(0, NVH, step=1, unroll=SUB)
            def norm(j):
                sl = pl.ds(j * L, L)
                s = sv[sl]
                bb = bv[sl]
                for t in range(4):
                    buf[ii[t], sl] = ((buf[ii[t], sl] - means[t])
                                      * rstds[t] * s + bb)

    def wait_gather(slot):
        pltpu.make_async_copy(word_hbm.at[idx_all.at[0]], bufs[slot],
                              gsems[slot]).wait()

    def wait_write0():
        pltpu.make_async_copy(buf0, out_hbm.at[pl.ds(0, CH)], ws0).wait()

    def wait_write1():
        pltpu.make_async_copy(buf1, out_hbm.at[pl.ds(0, CH)], ws1).wait()

    def chunk_off(c):
        b = c // QPC
        q = c - b * QPC
        return b * SEQ + pbase + q * CH, q * CH

    def pipe_body(k, _):
        c0 = 2 * k
        c1 = c0 + 1
        off0, poff0 = chunk_off(c0)
        off1, poff1 = chunk_off(c1)

        wait_gather(0)

        @pl.when(k > 0)
        def _():
            wait_write1()
        pltpu.async_copy(word_hbm.at[idx_all.at[c1]], buf1, gs1)

        compute_chunk(buf0, c0, poff0)
        pltpu.async_copy(buf0, out_hbm.at[pl.ds(off0, CH)], ws0)

        wait_gather(1)
        wait_write0()

        @pl.when(k < NCHUNK // 2 - 1)
        def _():
            c2 = c0 + 2
            off2, _p = chunk_off(c2)
            pltpu.async_copy(word_hbm.at[idx_all.at[c2]], buf0, gs0)

        compute_chunk(buf1, c1, poff1)
        pltpu.async_copy(buf1, out_hbm.at[pl.ds(off1, CH)], ws1)
        return 0

    lax.fori_loop(0, NCHUNK // 2, pipe_body, 0)
    wait_write1()


@jax.jit
def _sc_embed_ln(ids_flat, tt_flat, word_emb, pos_emb, type_emb, ln_scale,
                 ln_bias):
    mesh = plsc.VectorSubcoreMesh(core_axis_name="c", subcore_axis_name="s")
    f = functools.partial(
        pl.kernel,
        out_type=jax.ShapeDtypeStruct((BATCH * SEQ, HIDDEN), jnp.float32),
        mesh=mesh,
        scratch_types=[
            pltpu.VMEM((NCHUNK, CH), jnp.int32),
            pltpu.VMEM((TPW,), jnp.int32),
            pltpu.VMEM((CH, HIDDEN), jnp.float32),
            pltpu.VMEM((CH, HIDDEN), jnp.float32),
            pltpu.VMEM((2, PPW, HIDDEN), jnp.float32),
            pltpu.VMEM((2, HIDDEN), jnp.float32),
            pltpu.VMEM((HIDDEN,), jnp.float32),
            pltpu.VMEM((HIDDEN,), jnp.float32),
            pltpu.SemaphoreType.DMA,
            pltpu.SemaphoreType.DMA,
            pltpu.SemaphoreType.DMA,
            pltpu.SemaphoreType.DMA,
            pltpu.SemaphoreType.DMA,
        ],
    )(_sc_body)
    return f(ids_flat, tt_flat, word_emb, pos_emb, type_emb, ln_scale, ln_bias)


def kernel(input_ids, token_type_ids, word_emb, pos_emb, type_emb, ln_scale,
           ln_bias):
    b, s = input_ids.shape
    ids_flat = input_ids.reshape(-1).astype(jnp.int32)
    tt_flat = token_type_ids.reshape(-1).astype(jnp.int32)
    out = _sc_embed_ln(ids_flat, tt_flat, word_emb, pos_emb, type_emb,
                       ln_scale, ln_bias)
    return out.reshape(b, s, HIDDEN)


# speedup vs baseline: 1.1490x; 1.1490x over previous
import functools

import jax
import jax.numpy as jnp
from jax import lax
from jax.experimental import pallas as pl
from jax.experimental.pallas import tpu as pltpu
from jax.experimental.pallas import tpu_sc as plsc

VOCAB = 100000
HIDDEN = 768
BATCH = 4
SEQ = 2048
EPS = 1e-12

L = 16
NC, NS = 2, 16
NW = NC * NS
PPW = SEQ // NW
TPW = BATCH * PPW
CH = 16
NCHUNK = TPW // CH
QPC = PPW // CH
NVH = HIDDEN // L
SUB = 8

_DNUMS = lax.GatherDimensionNumbers(offset_dims=(), collapsed_slice_dims=(0,),
                                    start_index_map=(0,))


def _xlane_sum(v):
    idx = lax.iota(jnp.int32, L)
    for sh in (1, 2, 4, 8):
        perm = jnp.bitwise_xor(idx, sh)
        v = v + lax.gather(v, perm[:, None], _DNUMS, slice_sizes=(1,),
                           mode=lax.GatherScatterMode.PROMISE_IN_BOUNDS)
    return v


def _rsqrt(x):
    i = lax.bitcast_convert_type(x, jnp.int32)
    i = jnp.int32(0x5F3759DF) - lax.shift_right_arithmetic(i, 1)
    y = lax.bitcast_convert_type(i, jnp.float32)
    for _ in range(3):
        y = y * (1.5 - 0.5 * x * y * y)
    return y


def _sc_body(ids_hbm, tt_hbm, word_hbm, pos_hbm, typ_hbm, scale_hbm, bias_hbm,
             out_hbm, idx_all, tt_all, buf0, buf1, ptab, tbuf, sv, bv,
             ssem, gs0, gs1, ws0, ws1):
    wid = lax.axis_index("s") * NC + lax.axis_index("c")
    pbase = wid * PPW
    bufs = (buf0, buf1)
    gsems = (gs0, gs1)

    stage = []
    for b in range(BATCH):
        for q in range(QPC):
            c = QPC * b + q
            stage.append(pltpu.async_copy(
                ids_hbm.at[pl.ds(b * SEQ + pbase + q * CH, CH)],
                idx_all.at[c], ssem))
        stage.append(pltpu.async_copy(
            tt_hbm.at[pl.ds(b * SEQ + pbase, PPW)],
            tt_all.at[pl.ds(b * PPW, PPW)], ssem))
    stage.append(pltpu.async_copy(scale_hbm, sv, ssem))
    stage.append(pltpu.async_copy(bias_hbm, bv, ssem))
    stage.append(pltpu.async_copy(typ_hbm, tbuf, ssem))
    stage.append(pltpu.async_copy(pos_hbm.at[pl.ds(pbase, PPW)],
                                  ptab.at[0], ssem))
    for h in stage:
        h.wait()

    pltpu.async_copy(word_hbm.at[idx_all.at[0]], buf0, gs0)

    def prep_j(j, _):
        sl = pl.ds(j * L, L)
        t0 = tbuf[0, sl]
        d = tbuf[1, sl] - t0

        @plsc.parallel_loop(0, PPW, step=1, unroll=SUB)
        def prep_i(i):
            v0 = ptab[0, i, sl] + t0
            ptab[0, i, sl] = v0
            ptab[1, i, sl] = v0 + d
        return 0
    lax.fori_loop(0, NVH, prep_j, 0)

    zero = jnp.zeros((L,), jnp.float32)

    def compute_chunk(buf, c, poff):
        tt16 = tt_all[pl.ds(c * CH, CH)]
        for l in range(0, CH, 4):
            ii = [l + t for t in range(4)]
            prow = [poff + i for i in ii]
            ttk = [tt16[i] for i in ii]

            @plsc.parallel_loop(0, NVH, step=1, unroll=SUB,
                                carry=(zero,) * 8)
            def stats(j, carry):
                sl = pl.ds(j * L, L)
                out = []
                for t in range(4):
                    v = buf[ii[t], sl] + ptab[ttk[t], prow[t], sl]
                    buf[ii[t], sl] = v
                    out.append(carry[2 * t] + v)
                    out.append(carry[2 * t + 1] + v * v)
                return tuple(out)
            accs = stats

            means = [_xlane_sum(accs[2 * t]) * (1.0 / HIDDEN)
                     for t in range(4)]
            msqs = [_xlane_sum(accs[2 * t + 1]) * (1.0 / HIDDEN)
                    for t in range(4)]
            rstds = [_rsqrt(msqs[t] - means[t] * means[t] + EPS)
                     for t in range(4)]

            @plsc.parallel_loop(0, NVH, step=1, unroll=SUB)
            def norm(j):
                sl = pl.ds(j * L, L)
                s = sv[sl]
                bb = bv[sl]
                for t in range(4):
                    buf[ii[t], sl] = ((buf[ii[t], sl] - means[t])
                                      * rstds[t] * s + bb)

    def wait_gather(slot):
        pltpu.make_async_copy(word_hbm.at[idx_all.at[0]], bufs[slot],
                              gsems[slot]).wait()

    def wait_write0():
        pltpu.make_async_copy(buf0, out_hbm.at[pl.ds(0, CH)], ws0).wait()

    def wait_write1():
        pltpu.make_async_copy(buf1, out_hbm.at[pl.ds(0, CH)], ws1).wait()

    def chunk_off(c):
        b = c // QPC
        q = c - b * QPC
        return b * SEQ + pbase + q * CH, q * CH

    def pipe_body(k, _):
        c0 = 2 * k
        c1 = c0 + 1
        off0, poff0 = chunk_off(c0)
        off1, poff1 = chunk_off(c1)

        wait_gather(0)

        @pl.when(k > 0)
        def _():
            wait_write1()
        pltpu.async_copy(word_hbm.at[idx_all.at[c1]], buf1, gs1)

        compute_chunk(buf0, c0, poff0)
        pltpu.async_copy(buf0, out_hbm.at[pl.ds(off0, CH)], ws0)

        wait_gather(1)
        wait_write0()

        @pl.when(k < NCHUNK // 2 - 1)
        def _():
            c2 = c0 + 2
            off2, _p = chunk_off(c2)
            pltpu.async_copy(word_hbm.at[idx_all.at[c2]], buf0, gs0)

        compute_chunk(buf1, c1, poff1)
        pltpu.async_copy(buf1, out_hbm.at[pl.ds(off1, CH)], ws1)
        return 0

    lax.fori_loop(0, NCHUNK // 2, pipe_body, 0)
    wait_write1()


@jax.jit
def _sc_embed_ln(ids_flat, tt_flat, word_emb, pos_emb, type_emb, ln_scale,
                 ln_bias):
    mesh = plsc.VectorSubcoreMesh(core_axis_name="c", subcore_axis_name="s")
    f = functools.partial(
        pl.kernel,
        out_type=jax.ShapeDtypeStruct((BATCH * SEQ, HIDDEN), jnp.float32),
        mesh=mesh,
        scratch_types=[
            pltpu.VMEM((NCHUNK, CH), jnp.int32),
            pltpu.VMEM((TPW,), jnp.int32),
            pltpu.VMEM((CH, HIDDEN), jnp.float32),
            pltpu.VMEM((CH, HIDDEN), jnp.float32),
            pltpu.VMEM((2, PPW, HIDDEN), jnp.float32),
            pltpu.VMEM((2, HIDDEN), jnp.float32),
            pltpu.VMEM((HIDDEN,), jnp.float32),
            pltpu.VMEM((HIDDEN,), jnp.float32),
            pltpu.SemaphoreType.DMA,
            pltpu.SemaphoreType.DMA,
            pltpu.SemaphoreType.DMA,
            pltpu.SemaphoreType.DMA,
            pltpu.SemaphoreType.DMA,
        ],
    )(_sc_body)
    return f(ids_flat, tt_flat, word_emb, pos_emb, type_emb, ln_scale, ln_bias)


def kernel(input_ids, token_type_ids, word_emb, pos_emb, type_emb, ln_scale,
           ln_bias):
    b, s = input_ids.shape
    ids_flat = input_ids.reshape(-1).astype(jnp.int32)
    tt_flat = token_type_ids.reshape(-1).astype(jnp.int32)
    out = _sc_embed_ln(ids_flat, tt_flat, word_emb, pos_emb, type_emb,
                       ln_scale, ln_bias)
    return out.reshape(b, s, HIDDEN)
